# Initial kernel scaffold; baseline (speedup 1.0000x reference)
import jax
import jax.numpy as jnp
from jax.experimental import pallas as pl


def kernel(x, ln1_scale, ln1_bias, Wq, Wk, Wv, Wo, bo, ln2_scale, ln2_bias, Wr, W1, b1, W2, b2):
    raise NotImplementedError("write your pallas kernel here")


# speedup vs baseline: 1.4340x; 1.4340x over previous
import functools

import jax
import jax.numpy as jnp
from jax.experimental import pallas as pl
from jax.experimental.pallas import tpu as pltpu
from jax.experimental.pallas import tpu_sc as plsc

B_, N_, D_ = 2, 2048, 1024
H_ = 16
DH_ = D_ // H_
E_ = 8
K_ = 2
DFF_ = 4096
T_ = B_ * N_
CAP_ = 1280
TRASH_ = E_ * CAP_
NEG_ = -1.0

F32 = jnp.float32
BF16 = jnp.bfloat16


def _ln(xb, scale, bias, eps=1e-5):
    mu = jnp.mean(xb, axis=-1, keepdims=True)
    var = jnp.mean((xb - mu) ** 2, axis=-1, keepdims=True)
    return (xb - mu) / jnp.sqrt(var + eps) * scale + bias


def _ln_qkv_body(x_ref, s_ref, b_ref, wq_ref, wk_ref, wv_ref,
                 q_ref, k_ref, v_ref):
    xn = _ln(x_ref[...], s_ref[...], b_ref[...]).astype(BF16)
    q_ref[...] = jnp.dot(xn, wq_ref[...], preferred_element_type=F32).astype(BF16)
    k_ref[...] = jnp.dot(xn, wk_ref[...], preferred_element_type=F32).astype(BF16)
    v_ref[...] = jnp.dot(xn, wv_ref[...], preferred_element_type=F32).astype(BF16)


def _ln_qkv(xt, s, b, wq, wk, wv, bm=512):
    g = T_ // bm
    return pl.pallas_call(
        _ln_qkv_body,
        grid=(g,),
        in_specs=[
            pl.BlockSpec((bm, D_), lambda i: (i, 0)),
            pl.BlockSpec((1, D_), lambda i: (0, 0)),
            pl.BlockSpec((1, D_), lambda i: (0, 0)),
            pl.BlockSpec((D_, D_), lambda i: (0, 0)),
            pl.BlockSpec((D_, D_), lambda i: (0, 0)),
            pl.BlockSpec((D_, D_), lambda i: (0, 0)),
        ],
        out_specs=[pl.BlockSpec((bm, D_), lambda i: (i, 0))] * 3,
        out_shape=[jax.ShapeDtypeStruct((T_, D_), BF16)] * 3,
    )(xt, s, b, wq, wk, wv)


def _attn_body(q_ref, k_ref, v_ref, o_ref):
    q = q_ref[0, 0] * BF16(0.125)
    s = jax.lax.dot_general(q, k_ref[0, 0], (((1,), (1,)), ((), ())),
                            preferred_element_type=F32)
    m = jnp.max(s, axis=1, keepdims=True)
    p = jnp.exp(s - m)
    l = jnp.sum(p, axis=1, keepdims=True)
    o = jnp.dot(p.astype(BF16), v_ref[0, 0], preferred_element_type=F32) / l
    o_ref[0, 0] = o.astype(BF16)


def _attention(q4, k4, v4, bq=512):
    return pl.pallas_call(
        _attn_body,
        grid=(B_, H_, N_ // bq),
        in_specs=[
            pl.BlockSpec((1, 1, bq, DH_), lambda b, h, i: (b, h, i, 0)),
            pl.BlockSpec((1, 1, N_, DH_), lambda b, h, i: (b, h, 0, 0)),
            pl.BlockSpec((1, 1, N_, DH_), lambda b, h, i: (b, h, 0, 0)),
        ],
        out_specs=pl.BlockSpec((1, 1, bq, DH_), lambda b, h, i: (b, h, i, 0)),
        out_shape=jax.ShapeDtypeStruct((B_, H_, N_, DH_), BF16),
    )(q4, k4, v4)


def _postattn_body(o_ref, x_ref, wo_ref, bo_ref, s_ref, b_ref, wr_ref,
                   h_ref, xn_ref, lg_ref):
    ao = jnp.dot(o_ref[...], wo_ref[...], preferred_element_type=F32)
    h = x_ref[...] + ao + bo_ref[...]
    h_ref[...] = h
    xn = _ln(h, s_ref[...], b_ref[...])
    xn_ref[...] = xn
    lg_ref[...] = jnp.dot(xn.astype(BF16), wr_ref[...].astype(BF16),
                          preferred_element_type=F32)


def _postattn(o, xt, wo, bo, s, b, wr, bm=512):
    g = T_ // bm
    return pl.pallas_call(
        _postattn_body,
        grid=(g,),
        in_specs=[
            pl.BlockSpec((bm, D_), lambda i: (i, 0)),
            pl.BlockSpec((bm, D_), lambda i: (i, 0)),
            pl.BlockSpec((D_, D_), lambda i: (0, 0)),
            pl.BlockSpec((1, D_), lambda i: (0, 0)),
            pl.BlockSpec((1, D_), lambda i: (0, 0)),
            pl.BlockSpec((1, D_), lambda i: (0, 0)),
            pl.BlockSpec((D_, E_), lambda i: (0, 0)),
        ],
        out_specs=[
            pl.BlockSpec((bm, D_), lambda i: (i, 0)),
            pl.BlockSpec((bm, D_), lambda i: (i, 0)),
            pl.BlockSpec((bm, E_), lambda i: (i, 0)),
        ],
        out_shape=[
            jax.ShapeDtypeStruct((T_, D_), F32),
            jax.ShapeDtypeStruct((T_, D_), F32),
            jax.ShapeDtypeStruct((T_, E_), F32),
        ],
    )(o, xt, wo, bo, s, b, wr)


def _router_body(lg_ref, s0_ref, s1_ref, c0_ref, c1_ref, g0_ref, g1_ref,
                 carry):
    i = pl.program_id(0)
    bt = lg_ref.shape[0]

    @pl.when(i == 0)
    def _():
        carry[...] = jnp.zeros_like(carry)

    lg = lg_ref[...]
    m = jnp.max(lg, axis=1, keepdims=True)
    ex = jnp.exp(lg - m)
    p = ex / jnp.sum(ex, axis=1, keepdims=True)
    lane = jax.lax.broadcasted_iota(jnp.int32, (bt, E_), 1)
    v1 = jnp.max(p, axis=1, keepdims=True)
    i1 = jnp.min(jnp.where(p == v1, lane, E_), axis=1, keepdims=True)
    oh1 = lane == i1
    p2 = jnp.where(oh1, NEG_, p)
    v2 = jnp.max(p2, axis=1, keepdims=True)
    i2 = jnp.min(jnp.where(p2 == v2, lane, E_), axis=1, keepdims=True)
    oh2 = lane == i2

    a = oh1.astype(F32) + oh2.astype(F32)
    r = jax.lax.broadcasted_iota(jnp.int32, (bt, bt), 0)
    c = jax.lax.broadcasted_iota(jnp.int32, (bt, bt), 1)
    tri = (c < r).astype(BF16)
    pre = jnp.dot(tri, a.astype(BF16), preferred_element_type=F32) + carry[...]
    carry[...] = carry[...] + jnp.sum(a, axis=0, keepdims=True)

    pos0 = jnp.sum(pre * oh1, axis=1, keepdims=True).astype(jnp.int32)
    pos1 = jnp.sum(pre * oh2, axis=1, keepdims=True).astype(jnp.int32)
    ssum = v1 + v2
    keep0 = pos0 < CAP_
    keep1 = pos1 < CAP_
    s0_ref[...] = jnp.where(keep0, i1 * CAP_ + pos0, TRASH_)
    s1_ref[...] = jnp.where(keep1, i2 * CAP_ + pos1, TRASH_)
    c0_ref[...] = i1 * CAP_ + jnp.minimum(pos0, CAP_ - 1)
    c1_ref[...] = i2 * CAP_ + jnp.minimum(pos1, CAP_ - 1)
    g0_ref[...] = jnp.where(keep0, v1 / ssum, 0.0)
    g1_ref[...] = jnp.where(keep1, v2 / ssum, 0.0)


def _router(logits, bt=1024):
    g = T_ // bt
    return pl.pallas_call(
        _router_body,
        grid=(g,),
        in_specs=[pl.BlockSpec((bt, E_), lambda i: (i, 0))],
        out_specs=[pl.BlockSpec((bt, 1), lambda i: (i, 0))] * 6,
        out_shape=[jax.ShapeDtypeStruct((T_, 1), jnp.int32)] * 4
        + [jax.ShapeDtypeStruct((T_, 1), F32)] * 2,
        scratch_shapes=[pltpu.VMEM((1, E_), F32)],
    )(logits)


def _ffn_body(x_ref, w1_ref, b1_ref, w2_ref, b2_ref, o_ref):
    j = pl.program_id(1)
    s = jnp.dot(x_ref[...].astype(BF16), w1_ref[0].astype(BF16),
                preferred_element_type=F32) + b1_ref[0]
    hh = jax.nn.gelu(s).astype(BF16)
    part = jnp.dot(hh, w2_ref[0].astype(BF16), preferred_element_type=F32)

    @pl.when(j == 0)
    def _():
        o_ref[...] = part + b2_ref[0]

    @pl.when(j > 0)
    def _():
        o_ref[...] += part


def _ffn(buf, w1, b1, w2, b2, bf=1024):
    gj = DFF_ // bf
    return pl.pallas_call(
        _ffn_body,
        grid=(E_, gj),
        in_specs=[
            pl.BlockSpec((CAP_, D_), lambda e, j: (e, 0)),
            pl.BlockSpec((1, D_, bf), lambda e, j: (e, 0, j)),
            pl.BlockSpec((1, 1, bf), lambda e, j: (e, 0, j)),
            pl.BlockSpec((1, bf, D_), lambda e, j: (e, j, 0)),
            pl.BlockSpec((1, 1, D_), lambda e, j: (e, 0, 0)),
        ],
        out_specs=pl.BlockSpec((CAP_, D_), lambda e, j: (e, 0)),
        out_shape=jax.ShapeDtypeStruct((E_ * CAP_, D_), F32),
    )(buf, w1, b1.reshape(E_, 1, DFF_), w2, b2.reshape(E_, 1, D_))


def _combine_body(h_ref, a0_ref, a1_ref, g0_ref, g1_ref, o_ref):
    o_ref[...] = (h_ref[...] + g0_ref[...] * a0_ref[...]
                  + g1_ref[...] * a1_ref[...])


def _combine(h, gath, g0, g1, bm=512):
    g = T_ // bm
    return pl.pallas_call(
        _combine_body,
        grid=(g,),
        in_specs=[
            pl.BlockSpec((bm, D_), lambda i: (i, 0)),
            pl.BlockSpec((bm, D_), lambda i: (i, 0)),
            pl.BlockSpec((bm, D_), lambda i: (i + T_ // bm, 0)),
            pl.BlockSpec((bm, 1), lambda i: (i, 0)),
            pl.BlockSpec((bm, 1), lambda i: (i, 0)),
        ],
        out_specs=pl.BlockSpec((bm, D_), lambda i: (i, 0)),
        out_shape=jax.ShapeDtypeStruct((T_, D_), F32),
    )(h, gath, gath, g0, g1)


def _sc_mesh():
    return plsc.VectorSubcoreMesh(core_axis_name="c", subcore_axis_name="s")


_UNITS = 32


def _sc_dispatch(xn, s0, s1, ch=64):
    per = T_ // _UNITS

    @functools.partial(
        pl.kernel,
        out_type=jax.ShapeDtypeStruct(((E_ + 1) * CAP_, D_), F32),
        mesh=_sc_mesh(),
        scratch_types=[
            pltpu.VMEM((1, T_), jnp.int32),
            pltpu.VMEM((1, T_), jnp.int32),
            pltpu.VMEM((ch, D_), F32),
            pltpu.SemaphoreType.DMA,
        ],
    )
    def run(x_hbm, s0_hbm, s1_hbm, o_hbm, i0, i1, xbuf, sem):
        unit = jax.lax.axis_index("c") * 16 + jax.lax.axis_index("s")
        pltpu.async_copy(s0_hbm, i0, sem).wait()
        pltpu.async_copy(s1_hbm, i1, sem).wait()
        base = unit * per

        @pl.loop(0, per // ch)
        def _(j):
            st = base + j * ch
            pltpu.async_copy(x_hbm.at[pl.ds(st, ch), :], xbuf, sem).wait()
            pltpu.sync_copy(xbuf, o_hbm.at[i0.at[0, pl.ds(st, ch)]])
            pltpu.sync_copy(xbuf, o_hbm.at[i1.at[0, pl.ds(st, ch)]])

    return run(xn, s0, s1)


def _sc_gather(out_flat, cidx, ch=32):
    per = K_ * T_ // _UNITS

    @functools.partial(
        pl.kernel,
        out_type=jax.ShapeDtypeStruct((K_ * T_, D_), F32),
        mesh=_sc_mesh(),
        scratch_types=[
            pltpu.VMEM((1, K_ * T_), jnp.int32),
            pltpu.VMEM((ch, D_), F32),
            pltpu.SemaphoreType.DMA,
        ],
    )
    def run(data_hbm, i_hbm, o_hbm, ic, gbuf, sem):
        unit = jax.lax.axis_index("c") * 16 + jax.lax.axis_index("s")
        pltpu.async_copy(i_hbm, ic, sem).wait()
        base = unit * per

        @pl.loop(0, per // ch)
        def _(j):
            st = base + j * ch
            pltpu.sync_copy(data_hbm.at[ic.at[0, pl.ds(st, ch)]], gbuf)
            pltpu.async_copy(gbuf, o_hbm.at[pl.ds(st, ch), :], sem).wait()

    return run(out_flat, cidx)


def kernel(x, ln1_scale, ln1_bias, Wq, Wk, Wv, Wo, bo, ln2_scale, ln2_bias,
           Wr, W1, b1, W2, b2):
    xt = x.reshape(T_, D_)
    s1 = ln1_scale.reshape(1, D_)
    b1_ = ln1_bias.reshape(1, D_)
    s2 = ln2_scale.reshape(1, D_)
    b2_ = ln2_bias.reshape(1, D_)

    q, k, v = _ln_qkv(xt, s1, b1_, Wq.astype(BF16), Wk.astype(BF16),
                      Wv.astype(BF16))

    def heads(t):
        return t.reshape(B_, N_, H_, DH_).transpose(0, 2, 1, 3)

    o = _attention(heads(q), heads(k), heads(v))
    o = o.transpose(0, 2, 1, 3).reshape(T_, D_)
    h, xn2, logits = _postattn(o, xt, Wo.astype(BF16),
                               bo.reshape(1, D_), s2, b2_, Wr)
    sid0, sid1, cid0, cid1, g0, g1 = _router(logits)

    buf = _sc_dispatch(xn2, sid0.reshape(1, T_), sid1.reshape(1, T_))
    expert_out = _ffn(buf, W1, b1, W2, b2)
    cidx = jnp.concatenate([cid0.reshape(1, T_), cid1.reshape(1, T_)], axis=1)
    gath = _sc_gather(expert_out, cidx)
    out = _combine(h, gath, g0, g1)
    return out.reshape(B_, N_, D_)


# speedup vs baseline: 1.5806x; 1.1022x over previous
import functools

import jax
import jax.numpy as jnp
from jax.experimental import pallas as pl
from jax.experimental.pallas import tpu as pltpu
from jax.experimental.pallas import tpu_sc as plsc

B_, N_, D_ = 2, 2048, 1024
H_ = 16
DH_ = D_ // H_
E_ = 8
K_ = 2
DFF_ = 4096
T_ = B_ * N_
CAP_ = 1280
TRASH_ = E_ * CAP_
NEG_ = -1.0

F32 = jnp.float32
BF16 = jnp.bfloat16


def _ln(xb, scale, bias, eps=1e-5):
    mu = jnp.mean(xb, axis=-1, keepdims=True)
    var = jnp.mean((xb - mu) ** 2, axis=-1, keepdims=True)
    return (xb - mu) / jnp.sqrt(var + eps) * scale + bias


def _ln_qkv_body(x_ref, s_ref, b_ref, wq_ref, wk_ref, wv_ref,
                 q_ref, k_ref, v_ref):
    xn = _ln(x_ref[...], s_ref[...], b_ref[...]).astype(BF16)
    q_ref[...] = jnp.dot(xn, wq_ref[...], preferred_element_type=F32).astype(BF16)
    k_ref[...] = jnp.dot(xn, wk_ref[...], preferred_element_type=F32).astype(BF16)
    v_ref[...] = jnp.dot(xn, wv_ref[...], preferred_element_type=F32).astype(BF16)


def _ln_qkv(xt, s, b, wq, wk, wv, bm=512):
    g = T_ // bm
    return pl.pallas_call(
        _ln_qkv_body,
        grid=(g,),
        in_specs=[
            pl.BlockSpec((bm, D_), lambda i: (i, 0)),
            pl.BlockSpec((1, D_), lambda i: (0, 0)),
            pl.BlockSpec((1, D_), lambda i: (0, 0)),
            pl.BlockSpec((D_, D_), lambda i: (0, 0)),
            pl.BlockSpec((D_, D_), lambda i: (0, 0)),
            pl.BlockSpec((D_, D_), lambda i: (0, 0)),
        ],
        out_specs=[pl.BlockSpec((bm, D_), lambda i: (i, 0))] * 3,
        out_shape=[jax.ShapeDtypeStruct((T_, D_), BF16)] * 3,
    )(xt, s, b, wq, wk, wv)


def _attn_body(q_ref, k_ref, v_ref, o_ref):
    q = q_ref[0, 0] * BF16(0.125)
    s = jax.lax.dot_general(q, k_ref[0, 0], (((1,), (1,)), ((), ())),
                            preferred_element_type=F32)
    m = jnp.max(s, axis=1, keepdims=True)
    p = jnp.exp(s - m).astype(BF16)
    v_ext = jnp.concatenate(
        [v_ref[0, 0], jnp.ones((N_, DH_), BF16)], axis=1)
    ol = jnp.dot(p, v_ext, preferred_element_type=F32)
    o = ol[:, :DH_] / ol[:, DH_:DH_ + 1]
    o_ref[0, 0] = o.astype(BF16)


def _attention(q4, k4, v4, bq=1024):
    return pl.pallas_call(
        _attn_body,
        grid=(B_, H_, N_ // bq),
        in_specs=[
            pl.BlockSpec((1, 1, bq, DH_), lambda b, h, i: (b, h, i, 0)),
            pl.BlockSpec((1, 1, N_, DH_), lambda b, h, i: (b, h, 0, 0)),
            pl.BlockSpec((1, 1, N_, DH_), lambda b, h, i: (b, h, 0, 0)),
        ],
        out_specs=pl.BlockSpec((1, 1, bq, DH_), lambda b, h, i: (b, h, i, 0)),
        out_shape=jax.ShapeDtypeStruct((B_, H_, N_, DH_), BF16),
    )(q4, k4, v4)


def _postattn_body(o_ref, x_ref, wo_ref, bo_ref, s_ref, b_ref, wr_ref,
                   h_ref, xn_ref, lg_ref):
    ao = jnp.dot(o_ref[...], wo_ref[...], preferred_element_type=F32)
    h = x_ref[...] + ao + bo_ref[...]
    h_ref[...] = h
    xn = _ln(h, s_ref[...], b_ref[...])
    xn_ref[...] = xn
    lg_ref[...] = jnp.dot(xn.astype(BF16), wr_ref[...].astype(BF16),
                          preferred_element_type=F32)


def _postattn(o, xt, wo, bo, s, b, wr, bm=512):
    g = T_ // bm
    return pl.pallas_call(
        _postattn_body,
        grid=(g,),
        in_specs=[
            pl.BlockSpec((bm, D_), lambda i: (i, 0)),
            pl.BlockSpec((bm, D_), lambda i: (i, 0)),
            pl.BlockSpec((D_, D_), lambda i: (0, 0)),
            pl.BlockSpec((1, D_), lambda i: (0, 0)),
            pl.BlockSpec((1, D_), lambda i: (0, 0)),
            pl.BlockSpec((1, D_), lambda i: (0, 0)),
            pl.BlockSpec((D_, E_), lambda i: (0, 0)),
        ],
        out_specs=[
            pl.BlockSpec((bm, D_), lambda i: (i, 0)),
            pl.BlockSpec((bm, D_), lambda i: (i, 0)),
            pl.BlockSpec((bm, E_), lambda i: (i, 0)),
        ],
        out_shape=[
            jax.ShapeDtypeStruct((T_, D_), F32),
            jax.ShapeDtypeStruct((T_, D_), F32),
            jax.ShapeDtypeStruct((T_, E_), F32),
        ],
    )(o, xt, wo, bo, s, b, wr)


def _router_body(lg_ref, s0_ref, s1_ref, c0_ref, c1_ref, g0_ref, g1_ref,
                 carry):
    i = pl.program_id(0)
    bt = lg_ref.shape[0]

    @pl.when(i == 0)
    def _():
        carry[...] = jnp.zeros_like(carry)

    lg = lg_ref[...]
    m = jnp.max(lg, axis=1, keepdims=True)
    ex = jnp.exp(lg - m)
    p = ex / jnp.sum(ex, axis=1, keepdims=True)
    lane = jax.lax.broadcasted_iota(jnp.int32, (bt, E_), 1)
    v1 = jnp.max(p, axis=1, keepdims=True)
    i1 = jnp.min(jnp.where(p == v1, lane, E_), axis=1, keepdims=True)
    oh1 = lane == i1
    p2 = jnp.where(oh1, NEG_, p)
    v2 = jnp.max(p2, axis=1, keepdims=True)
    i2 = jnp.min(jnp.where(p2 == v2, lane, E_), axis=1, keepdims=True)
    oh2 = lane == i2

    a = oh1.astype(F32) + oh2.astype(F32)
    r = jax.lax.broadcasted_iota(jnp.int32, (bt, bt), 0)
    c = jax.lax.broadcasted_iota(jnp.int32, (bt, bt), 1)
    tri = (c < r).astype(BF16)
    pre = jnp.dot(tri, a.astype(BF16), preferred_element_type=F32) + carry[...]
    carry[...] = carry[...] + jnp.sum(a, axis=0, keepdims=True)

    pos0 = jnp.sum(pre * oh1, axis=1, keepdims=True).astype(jnp.int32)
    pos1 = jnp.sum(pre * oh2, axis=1, keepdims=True).astype(jnp.int32)
    ssum = v1 + v2
    keep0 = pos0 < CAP_
    keep1 = pos1 < CAP_
    s0_ref[...] = jnp.where(keep0, i1 * CAP_ + pos0, TRASH_)
    s1_ref[...] = jnp.where(keep1, i2 * CAP_ + pos1, TRASH_)
    c0_ref[...] = i1 * CAP_ + jnp.minimum(pos0, CAP_ - 1)
    c1_ref[...] = i2 * CAP_ + jnp.minimum(pos1, CAP_ - 1)
    g0_ref[...] = jnp.where(keep0, v1 / ssum, 0.0)
    g1_ref[...] = jnp.where(keep1, v2 / ssum, 0.0)


def _router(logits, bt=1024):
    g = T_ // bt
    return pl.pallas_call(
        _router_body,
        grid=(g,),
        in_specs=[pl.BlockSpec((bt, E_), lambda i: (i, 0))],
        out_specs=[pl.BlockSpec((bt, 1), lambda i: (i, 0))] * 6,
        out_shape=[jax.ShapeDtypeStruct((T_, 1), jnp.int32)] * 4
        + [jax.ShapeDtypeStruct((T_, 1), F32)] * 2,
        scratch_shapes=[pltpu.VMEM((1, E_), F32)],
    )(logits)


def _ffn_body(x_ref, w1_ref, b1_ref, w2_ref, b2_ref, o_ref, xb_ref):
    j = pl.program_id(1)

    @pl.when(j == 0)
    def _():
        xb_ref[...] = x_ref[...].astype(BF16)

    s = jnp.dot(xb_ref[...], w1_ref[0].astype(BF16),
                preferred_element_type=F32) + b1_ref[0]
    hh = jax.nn.gelu(s.astype(BF16))
    part = jnp.dot(hh, w2_ref[0].astype(BF16), preferred_element_type=F32)

    @pl.when(j == 0)
    def _():
        o_ref[...] = part + b2_ref[0]

    @pl.when(j > 0)
    def _():
        o_ref[...] += part


def _ffn(buf, w1, b1, w2, b2, bf=1024):
    gj = DFF_ // bf
    return pl.pallas_call(
        _ffn_body,
        grid=(E_, gj),
        in_specs=[
            pl.BlockSpec((CAP_, D_), lambda e, j: (e, 0)),
            pl.BlockSpec((1, D_, bf), lambda e, j: (e, 0, j)),
            pl.BlockSpec((1, 1, bf), lambda e, j: (e, 0, j)),
            pl.BlockSpec((1, bf, D_), lambda e, j: (e, j, 0)),
            pl.BlockSpec((1, 1, D_), lambda e, j: (e, 0, 0)),
        ],
        out_specs=pl.BlockSpec((CAP_, D_), lambda e, j: (e, 0)),
        out_shape=jax.ShapeDtypeStruct((E_ * CAP_, D_), F32),
        scratch_shapes=[pltpu.VMEM((CAP_, D_), BF16)],
    )(buf, w1, b1.reshape(E_, 1, DFF_), w2, b2.reshape(E_, 1, D_))


def _combine_body(h_ref, a0_ref, a1_ref, g0_ref, g1_ref, o_ref):
    o_ref[...] = (h_ref[...] + g0_ref[...] * a0_ref[...]
                  + g1_ref[...] * a1_ref[...])


def _combine(h, gath, g0, g1, bm=512):
    g = T_ // bm
    return pl.pallas_call(
        _combine_body,
        grid=(g,),
        in_specs=[
            pl.BlockSpec((bm, D_), lambda i: (i, 0)),
            pl.BlockSpec((bm, D_), lambda i: (i, 0)),
            pl.BlockSpec((bm, D_), lambda i: (i + T_ // bm, 0)),
            pl.BlockSpec((bm, 1), lambda i: (i, 0)),
            pl.BlockSpec((bm, 1), lambda i: (i, 0)),
        ],
        out_specs=pl.BlockSpec((bm, D_), lambda i: (i, 0)),
        out_shape=jax.ShapeDtypeStruct((T_, D_), F32),
    )(h, gath, gath, g0, g1)


def _sc_mesh():
    return plsc.VectorSubcoreMesh(core_axis_name="c", subcore_axis_name="s")


_UNITS = 32


def _sc_dispatch(xn, s0, s1, ch=64):
    per = T_ // _UNITS

    @functools.partial(
        pl.kernel,
        out_type=jax.ShapeDtypeStruct(((E_ + 1) * CAP_, D_), F32),
        mesh=_sc_mesh(),
        scratch_types=[
            pltpu.VMEM((1, T_), jnp.int32),
            pltpu.VMEM((1, T_), jnp.int32),
            pltpu.VMEM((ch, D_), F32),
            pltpu.SemaphoreType.DMA,
        ],
    )
    def run(x_hbm, s0_hbm, s1_hbm, o_hbm, i0, i1, xbuf, sem):
        unit = jax.lax.axis_index("c") * 16 + jax.lax.axis_index("s")
        pltpu.async_copy(s0_hbm, i0, sem).wait()
        pltpu.async_copy(s1_hbm, i1, sem).wait()
        base = unit * per

        @pl.loop(0, per // ch)
        def _(j):
            st = base + j * ch
            pltpu.async_copy(x_hbm.at[pl.ds(st, ch), :], xbuf, sem).wait()
            pltpu.sync_copy(xbuf, o_hbm.at[i0.at[0, pl.ds(st, ch)]])
            pltpu.sync_copy(xbuf, o_hbm.at[i1.at[0, pl.ds(st, ch)]])

    return run(xn, s0, s1)


def _sc_gather(out_flat, cidx, ch=32):
    per = K_ * T_ // _UNITS

    @functools.partial(
        pl.kernel,
        out_type=jax.ShapeDtypeStruct((K_ * T_, D_), F32),
        mesh=_sc_mesh(),
        scratch_types=[
            pltpu.VMEM((1, K_ * T_), jnp.int32),
            pltpu.VMEM((ch, D_), F32),
            pltpu.SemaphoreType.DMA,
        ],
    )
    def run(data_hbm, i_hbm, o_hbm, ic, gbuf, sem):
        unit = jax.lax.axis_index("c") * 16 + jax.lax.axis_index("s")
        pltpu.async_copy(i_hbm, ic, sem).wait()
        base = unit * per

        @pl.loop(0, per // ch)
        def _(j):
            st = base + j * ch
            pltpu.sync_copy(data_hbm.at[ic.at[0, pl.ds(st, ch)]], gbuf)
            pltpu.async_copy(gbuf, o_hbm.at[pl.ds(st, ch), :], sem).wait()

    return run(out_flat, cidx)


def kernel(x, ln1_scale, ln1_bias, Wq, Wk, Wv, Wo, bo, ln2_scale, ln2_bias,
           Wr, W1, b1, W2, b2):
    xt = x.reshape(T_, D_)
    s1 = ln1_scale.reshape(1, D_)
    b1_ = ln1_bias.reshape(1, D_)
    s2 = ln2_scale.reshape(1, D_)
    b2_ = ln2_bias.reshape(1, D_)

    q, k, v = _ln_qkv(xt, s1, b1_, Wq.astype(BF16), Wk.astype(BF16),
                      Wv.astype(BF16))

    def heads(t):
        return t.reshape(B_, N_, H_, DH_).transpose(0, 2, 1, 3)

    o = _attention(heads(q), heads(k), heads(v))
    o = o.transpose(0, 2, 1, 3).reshape(T_, D_)
    h, xn2, logits = _postattn(o, xt, Wo.astype(BF16),
                               bo.reshape(1, D_), s2, b2_, Wr)
    sid0, sid1, cid0, cid1, g0, g1 = _router(logits)

    buf = _sc_dispatch(xn2, sid0.reshape(1, T_), sid1.reshape(1, T_))
    expert_out = _ffn(buf, W1, b1, W2, b2)
    cidx = jnp.concatenate([cid0.reshape(1, T_), cid1.reshape(1, T_)], axis=1)
    gath = _sc_gather(expert_out, cidx)
    out = _combine(h, gath, g0, g1)
    return out.reshape(B_, N_, D_)
